# baseline (speedup 1.0000x reference)
import jax
import jax.numpy as jnp
from jax.experimental import pallas as pl
from jax.experimental.pallas import tpu as pltpu
from jax.experimental.pallas import tpu_sc as plsc

EMBED_DIM = 64
RANK_DIM = 8
SCALING = 16.0 / 8.0
GATHER_WINDOW = 128
TC_BLOCK_ROWS = 2048


def _sc_gather(table_wide, idx_half):
    n = idx_half.shape[0]
    width = table_wide.shape[1]
    indices = idx_half.reshape(1, n)
    mesh = plsc.VectorSubcoreMesh(core_axis_name="core",
                                  subcore_axis_name="subcore")

    @pl.kernel(out_type=jax.ShapeDtypeStruct((n, width), table_wide.dtype),
               mesh=mesh)
    def gather_kernel(tab_hbm, i_hbm, o_hbm):
        def body(i_vmem, o_vmem):
            pltpu.sync_copy(tab_hbm.at[i_vmem.at[0]], o_vmem)

        pltpu.emit_pipeline(
            body,
            grid=(n // GATHER_WINDOW,),
            in_specs=[pl.BlockSpec((1, GATHER_WINDOW), lambda i: (0, i))],
            out_specs=[pl.BlockSpec((GATHER_WINDOW, width),
                                    lambda i: (i, 0))],
            core_axis_name=("core", "subcore"),
            dimension_semantics=(pltpu.PARALLEL,),
        )(i_hbm, o_hbm)

    return gather_kernel(table_wide, indices)


def _tc_select_lora(g_wide, idx, a_t, b_t):
    n = g_wide.shape[0]
    idx2 = idx.reshape(n, 1)

    def body(g_ref, i_ref, at_ref, bt_ref, o_ref):
        parity = (i_ref[...] & 1).astype(jnp.float32)
        gb = g_ref[...]
        sel = gb[:, :EMBED_DIM] * (1.0 - parity) + gb[:, EMBED_DIM:] * parity
        w_t = jnp.dot(at_ref[...], bt_ref[...],
                      preferred_element_type=jnp.float32)
        o_ref[...] = sel + jnp.dot(sel, w_t,
                                   preferred_element_type=jnp.float32) * SCALING

    return pl.pallas_call(
        body,
        grid=(n // TC_BLOCK_ROWS,),
        in_specs=[
            pl.BlockSpec((TC_BLOCK_ROWS, 2 * EMBED_DIM), lambda i: (i, 0)),
            pl.BlockSpec((TC_BLOCK_ROWS, 1), lambda i: (i, 0)),
            pl.BlockSpec((EMBED_DIM, RANK_DIM), lambda i: (0, 0)),
            pl.BlockSpec((RANK_DIM, EMBED_DIM), lambda i: (0, 0)),
        ],
        out_specs=pl.BlockSpec((TC_BLOCK_ROWS, EMBED_DIM), lambda i: (i, 0)),
        out_shape=jax.ShapeDtypeStruct((n, EMBED_DIM), jnp.float32),
    )(g_wide, idx2, a_t, b_t)


def kernel(x, table, lora_A, lora_B):
    bsz, seq = x.shape
    idx = x.reshape(-1).astype(jnp.int32)
    table_wide = table.reshape(table.shape[0] // 2, 2 * EMBED_DIM)
    g_wide = _sc_gather(table_wide, idx >> 1)
    out = _tc_select_lora(g_wide, idx, lora_A.T, lora_B.T)
    return out.reshape(bsz, seq, EMBED_DIM)


# speedup vs baseline: 1.0082x; 1.0082x over previous
import jax
import jax.numpy as jnp
from jax.experimental import pallas as pl
from jax.experimental.pallas import tpu as pltpu
from jax.experimental.pallas import tpu_sc as plsc

EMBED_DIM = 64
RANK_DIM = 8
SCALING = 16.0 / 8.0
GATHER_WINDOW = 128
TC_BLOCK_ROWS = 2048


def _sc_gather(table_wide, idx_half):
    n = idx_half.shape[0]
    width = table_wide.shape[1]
    indices = idx_half.reshape(1, n)
    mesh = plsc.VectorSubcoreMesh(core_axis_name="core",
                                  subcore_axis_name="subcore")

    @pl.kernel(out_type=jax.ShapeDtypeStruct((n, width), table_wide.dtype),
               mesh=mesh)
    def gather_kernel(tab_hbm, i_hbm, o_hbm):
        def body(i_vmem, o_vmem):
            pltpu.sync_copy(tab_hbm.at[i_vmem.at[0]], o_vmem)

        pltpu.emit_pipeline(
            body,
            grid=(n // GATHER_WINDOW,),
            in_specs=[pl.BlockSpec((1, GATHER_WINDOW), lambda i: (0, i))],
            out_specs=[pl.BlockSpec((GATHER_WINDOW, width),
                                    lambda i: (i, 0))],
            core_axis_name=("core", "subcore"),
            dimension_semantics=(pltpu.PARALLEL,),
        )(i_hbm, o_hbm)

    return gather_kernel(table_wide, indices)


def _tc_select_lora(g_wide, par_t, a_t, b_t):
    n = g_wide.shape[0]
    par_cols = TC_BLOCK_ROWS // 128

    def body(g_ref, p_ref, at_ref, bt_ref, o_ref):
        gb = g_ref[...]
        left = gb[:, :EMBED_DIM]
        diff = gb[:, EMBED_DIM:] - left
        parts = []
        for j in range(par_cols):
            lo, hi = j * 128, (j + 1) * 128
            parts.append(left[lo:hi] + diff[lo:hi] * p_ref[0, :, j:j + 1])
        sel = jnp.concatenate(parts, axis=0)
        w_t = jnp.dot(at_ref[...], bt_ref[...],
                      preferred_element_type=jnp.float32)
        o_ref[...] = sel + jnp.dot(sel, w_t,
                                   preferred_element_type=jnp.float32) * SCALING

    return pl.pallas_call(
        body,
        grid=(n // TC_BLOCK_ROWS,),
        in_specs=[
            pl.BlockSpec((TC_BLOCK_ROWS, 2 * EMBED_DIM), lambda i: (i, 0)),
            pl.BlockSpec((1, 128, par_cols), lambda i: (i, 0, 0)),
            pl.BlockSpec((EMBED_DIM, RANK_DIM), lambda i: (0, 0)),
            pl.BlockSpec((RANK_DIM, EMBED_DIM), lambda i: (0, 0)),
        ],
        out_specs=pl.BlockSpec((TC_BLOCK_ROWS, EMBED_DIM), lambda i: (i, 0)),
        out_shape=jax.ShapeDtypeStruct((n, EMBED_DIM), jnp.float32),
    )(g_wide, par_t, a_t, b_t)


def kernel(x, table, lora_A, lora_B):
    bsz, seq = x.shape
    n = bsz * seq
    idx = x.reshape(-1).astype(jnp.int32)
    par_cols = TC_BLOCK_ROWS // 128
    par_t = ((idx & 1).astype(jnp.float32)
             .reshape(n // 128, 128).T
             .reshape(128, n // TC_BLOCK_ROWS, par_cols)
             .transpose(1, 0, 2))
    table_wide = table.reshape(table.shape[0] // 2, 2 * EMBED_DIM)
    g_wide = _sc_gather(table_wide, idx >> 1)
    out = _tc_select_lora(g_wide, par_t, lora_A.T, lora_B.T)
    return out.reshape(bsz, seq, EMBED_DIM)
